# baseline (speedup 1.0000x reference)
import functools

import jax
import jax.numpy as jnp
from jax import lax
from jax.experimental import pallas as pl
from jax.experimental.pallas import tpu as pltpu
from jax.experimental.pallas import tpu_sc as plsc

N = 10000
E = 320000
D = 128

NW = 32
NPT = 320
N_PAD = NW * NPT

CH = 3200
FLUSH = 32768
BUF = FLUSH + CH + 16
K = 128
K4 = 256
NEG = -3.0e38

_mesh = plsc.VectorSubcoreMesh(
    core_axis_name="c", subcore_axis_name="s", num_cores=2, num_subcores=16
)
_SC_PARAMS = pltpu.CompilerParams(needs_layout_passes=False)


def _wid():
    return lax.axis_index("s") * 2 + lax.axis_index("c")


@functools.partial(
    pl.kernel,
    out_type=(
        jax.ShapeDtypeStruct((NW * E,), jnp.int32),
        jax.ShapeDtypeStruct((NW * 16,), jnp.int32),
    ),
    mesh=_mesh,
    compiler_params=_SC_PARAMS,
    scratch_types=[
        pltpu.VMEM((CH,), jnp.int32),
        pltpu.VMEM((CH,), jnp.int32),
        pltpu.VMEM((CH,), jnp.int32),
        pltpu.VMEM((CH,), jnp.int32),
        pltpu.VMEM((BUF,), jnp.int32),
        pltpu.VMEM((16,), jnp.int32),
        pltpu.SemaphoreType.DMA,
        pltpu.SemaphoreType.DMA,
    ],
)
def _bucket_kernel(src_hbm, dst_hbm, buckets_hbm, counts_hbm,
                   sbuf0, sbuf1, dbuf0, dbuf1, obuf, cntv, sem0, sem1):
    wid = _wid()
    lo = wid * NPT
    hi = lo + NPT
    one16 = jnp.full((16,), 1, jnp.int32)
    zero16 = jnp.full((16,), 0, jnp.int32)
    sems = (sem0, sem1)
    sbufs = (sbuf0, sbuf1)
    dbufs = (dbuf0, dbuf1)
    NCH = E // CH

    def start_load(ch, b):
        base = pl.multiple_of(ch * CH, 8)
        pltpu.async_copy(src_hbm.at[pl.ds(base, CH)], sbufs[b], sems[b])
        pltpu.async_copy(dst_hbm.at[pl.ds(base, CH)], dbufs[b], sems[b])

    def wait_load(ch, b):
        base = pl.multiple_of(ch * CH, 8)
        pltpu.make_async_copy(src_hbm.at[pl.ds(base, CH)], sbufs[b],
                              sems[b]).wait()
        pltpu.make_async_copy(dst_hbm.at[pl.ds(base, CH)], dbufs[b],
                              sems[b]).wait()

    start_load(0, 0)

    def do_chunk(ch, b, carry):
        @pl.when(ch + 1 < NCH)
        def _():
            start_load(ch + 1, 1 - b)

        wait_load(ch, b)

        def group_body(gg, carry):
            cnt, off = carry
            for u in range(4):
                g = gg * 4 + u
                d16 = dbufs[b][pl.ds(g * 16, 16)]
                s16 = sbufs[b][pl.ds(g * 16, 16)]
                m = (d16 >= lo) & (d16 < hi)
                mi = jnp.where(m, one16, zero16)
                ent = (s16 << 9) | (d16 - lo)
                cs = plsc.cumsum(mi)
                pos = cnt + cs - 1
                plsc.store_scatter(obuf, [pos], ent, mask=m)
                cnt = cnt + cs[15]
            return cnt, off

        cnt, off = lax.fori_loop(0, CH // 64, group_body, carry)

        do_flush = cnt >= FLUSH

        @pl.when(do_flush)
        def _():
            pltpu.sync_copy(
                obuf.at[pl.ds(0, FLUSH)],
                buckets_hbm.at[pl.ds(pl.multiple_of(wid * E + off, 8),
                                     FLUSH)],
            )
            nmv = (cnt - FLUSH + 15) >> 4

            def mv_body(i, _):
                obuf[pl.ds(i * 16, 16)] = obuf[pl.ds(FLUSH + i * 16, 16)]
                return 0

            lax.fori_loop(0, nmv, mv_body, 0)

        cnt = jnp.where(do_flush, cnt - FLUSH, cnt)
        off = jnp.where(do_flush, off + FLUSH, off)
        return cnt, off

    def pair_body(p, carry):
        for b in range(2):
            carry = do_chunk(p * 2 + b, b, carry)
        return carry

    cnt, off = lax.fori_loop(
        0, NCH // 2, pair_body, (jnp.int32(0), jnp.int32(0))
    )

    nblk = (cnt + 511) >> 9

    def blk_body(b, _):
        pltpu.sync_copy(
            obuf.at[pl.ds(pl.multiple_of(b * 512, 8), 512)],
            buckets_hbm.at[pl.ds(pl.multiple_of(wid * E + off + b * 512, 8),
                                 512)],
        )
        return 0

    lax.fori_loop(0, nblk, blk_body, 0)
    cntv[...] = jnp.full((16,), 0, jnp.int32) + (off + cnt)
    pltpu.sync_copy(cntv, counts_hbm.at[pl.ds(pl.multiple_of(wid * 16, 8),
                                              16)])


DW = D // 2
NPP = NPT // 2

@functools.partial(
    pl.kernel,
    out_type=jax.ShapeDtypeStruct((N_PAD // 2, D), jnp.int32),
    mesh=_mesh,
    compiler_params=_SC_PARAMS,
    scratch_types=[
        pltpu.VMEM((K,), jnp.int32),
        pltpu.VMEM((K,), jnp.int32),
        pltpu.VMEM((K,), jnp.int32),
        pltpu.VMEM((K,), jnp.int32),
        pltpu.VMEM((K,), jnp.int32),
        pltpu.VMEM((K,), jnp.int32),
        pltpu.VMEM((K,), jnp.int32),
        pltpu.VMEM((K, D), jnp.int32),
        pltpu.VMEM((K, D), jnp.int32),
        pltpu.VMEM((NPT + 8, DW), jnp.int32),
        pltpu.VMEM((16,), jnp.int32),
        pltpu.SemaphoreType.DMA,
        pltpu.SemaphoreType.DMA,
    ],
)
def _segmax_kernel(y_hbm, buckets_hbm, counts_hbm, aggr_hbm,
                   pk, sidx0, sidx1, dloc0, dloc1,
                   soff0, soff1, rows0, rows1, acc, cntv, sem0, sem1):
    wid = _wid()
    lop = pl.multiple_of(wid * NPP, 8)
    iota = lax.iota(jnp.int32, 16)
    sems = (sem0, sem1)
    sidxs = (sidx0, sidx1)
    dlocs = (dloc0, dloc1)
    soffs = (soff0, soff1)
    rowss = (rows0, rows1)
    H = NPP // 2

    for t in range(2):
        pltpu.sync_copy(y_hbm.at[pl.ds(lop + t * H, H)],
                        rows0.at[pl.ds(0, H)])

        def init_body(p, _):
            for par in range(2):
                for c in range(DW // 16):
                    acc[t * 2 * H + p * 2 + par, pl.ds(c * 16, 16)] = (
                        rows0[p, pl.ds(par * DW + c * 16, 16)])
            return 0

        lax.fori_loop(0, H, init_body, 0)

    pltpu.sync_copy(counts_hbm.at[pl.ds(pl.multiple_of(wid * 16, 8), 16)],
                    cntv)
    cnt = cntv[...][0]
    nchunks = (cnt + (K - 1)) // K

    def start_chunk(ch, b):
        base = ch * K
        pltpu.sync_copy(
            buckets_hbm.at[pl.ds(pl.multiple_of(wid * E + base, 8), K)], pk)
        for g in range(K // 16):
            e = pk[pl.ds(g * 16, 16)]
            valid = (base + g * 16 + iota) < cnt
            src = jnp.where(valid, e >> 10, jnp.full((16,), 0, jnp.int32))
            so = ((e >> 9) & 1) << 6
            dl = jnp.where(valid, e & 511, jnp.full((16,), NPT, jnp.int32))
            sl16 = pl.ds(g * 16, 16)
            sidxs[b][sl16] = src
            soffs[b][sl16] = so
            dlocs[b][sl16] = dl
        pltpu.async_copy(y_hbm.at[sidxs[b]], rowss[b], sems[b])

    @pl.when(nchunks > 0)
    def _():
        start_chunk(0, 0)

    def do_chunk(ch, b):
        @pl.when(ch + 1 < nchunks)
        def _():
            start_chunk(ch + 1, 1 - b)

        @pl.when(ch < nchunks)
        def _():
            pltpu.make_async_copy(y_hbm.at[sidxs[b]], rowss[b],
                                  sems[b]).wait()
            for g in range(K // 16):
                sl16 = pl.ds(g * 16, 16)
                dl16 = dlocs[b][sl16]
                so16 = soffs[b][sl16]
                for lane in range(16):
                    d = dl16[lane]
                    so = so16[lane]
                    ei = g * 16 + lane
                    for c in range(DW // 16):
                        sla = pl.ds(c * 16, 16)
                        a = plsc.bitcast(acc[d, sla], jnp.bfloat16)
                        r = plsc.bitcast(
                            rowss[b][ei, pl.ds(so + c * 16, 16)],
                            jnp.bfloat16)
                        acc[d, sla] = plsc.bitcast(jnp.maximum(a, r),
                                                   jnp.int32)

    def pair_body(p, _):
        do_chunk(p * 2, 0)
        do_chunk(p * 2 + 1, 1)
        return 0

    lax.fori_loop(0, (nchunks + 1) // 2, pair_body, 0)

    for t in range(2):
        def wb_body(p, _):
            for par in range(2):
                for c in range(DW // 16):
                    rows0[p, pl.ds(par * DW + c * 16, 16)] = (
                        acc[t * 2 * H + p * 2 + par, pl.ds(c * 16, 16)])
            return 0

        lax.fori_loop(0, H, wb_body, 0)
        pltpu.sync_copy(rows0.at[pl.ds(0, H)],
                        aggr_hbm.at[pl.ds(lop + t * H, H)])


@functools.partial(
    pl.kernel,
    out_type=jax.ShapeDtypeStruct((N_PAD,), jnp.float32),
    mesh=_mesh,
    compiler_params=_SC_PARAMS,
    scratch_types=[
        pltpu.VMEM((N_PAD,), jnp.float32),
        pltpu.VMEM((K4,), jnp.int32),
        pltpu.VMEM((16, 512), jnp.float32),
        pltpu.VMEM((NPT,), jnp.float32),
        pltpu.VMEM((16,), jnp.int32),
    ],
)
def _segmax1_kernel(y4_hbm, buckets_hbm, counts_hbm, aggr_hbm,
                    y4v, pk, acc2, outv, cntv):
    wid = _wid()
    lo = pl.multiple_of(wid * NPT, 8)
    iota = lax.iota(jnp.int32, 16)
    pltpu.sync_copy(y4_hbm, y4v)
    pltpu.sync_copy(counts_hbm.at[pl.ds(pl.multiple_of(wid * 16, 8), 16)],
                    cntv)
    cnt = cntv[...][0]
    neg = jnp.full((16,), NEG, jnp.float32)
    for r in range(16):
        for c in range(512 // 16):
            acc2[r, pl.ds(c * 16, 16)] = neg
    nchunks = (cnt + (K4 - 1)) // K4

    def chunk(ch, _):
        base = ch * K4
        pltpu.sync_copy(
            buckets_hbm.at[pl.ds(pl.multiple_of(wid * E + base, 8), K4)], pk)
        for g in range(K4 // 16):
            e = pk[pl.ds(g * 16, 16)]
            valid = (base + g * 16 + iota) < cnt
            src = jnp.where(valid, e >> 9, jnp.full((16,), 0, jnp.int32))
            dl = e & 511
            vals = plsc.load_gather(y4v, [src])
            cur = plsc.load_gather(acc2, [iota, dl])
            plsc.store_scatter(
                acc2, [iota, dl], jnp.maximum(vals, cur), mask=valid
            )
        return 0

    lax.fori_loop(0, nchunks, chunk, 0)
    for c in range(NPT // 16):
        v = y4v[pl.ds(lo + c * 16, 16)]
        for r in range(16):
            v = jnp.maximum(v, acc2[r, pl.ds(c * 16, 16)])
        outv[pl.ds(c * 16, 16)] = v
    pltpu.sync_copy(outv, aggr_hbm.at[pl.ds(lo, NPT)])


R = 1024
_DN = (((1,), (1,)), ((), ()))


def _lin_body(h_ref, w_ref, b_ref, y_ref):
    y_ref[...] = jnp.maximum(
        lax.dot_general(h_ref[...], w_ref[...], _DN,
                        preferred_element_type=jnp.float32) + b_ref[...],
        0.0,
    ).astype(jnp.bfloat16)


def _lin(h, W, b):
    return pl.pallas_call(
        _lin_body,
        grid=(N_PAD // R,),
        in_specs=[
            pl.BlockSpec((R, D), lambda i: (i, 0)),
            pl.BlockSpec((D, D), lambda i: (0, 0)),
            pl.BlockSpec((1, D), lambda i: (0, 0)),
        ],
        out_specs=pl.BlockSpec((R, D), lambda i: (i, 0)),
        out_shape=jax.ShapeDtypeStruct((N_PAD, D), jnp.bfloat16),
    )(h, W, b.reshape(1, D))


def _upd_body(a_ref, h_ref, u1_ref, u2_ref, w_ref, b_ref, hn_ref, yn_ref):
    hn = jnp.maximum(
        lax.dot_general(a_ref[...].astype(jnp.float32), u1_ref[...], _DN,
                        preferred_element_type=jnp.float32)
        + lax.dot_general(h_ref[...], u2_ref[...], _DN,
                          preferred_element_type=jnp.float32),
        0.0,
    )
    hn_ref[...] = hn
    yn_ref[...] = jnp.maximum(
        lax.dot_general(hn, w_ref[...], _DN,
                        preferred_element_type=jnp.float32) + b_ref[...],
        0.0,
    ).astype(jnp.bfloat16)


def _upd(a, h, U1, U2, Wn, bn):
    return pl.pallas_call(
        _upd_body,
        grid=(N_PAD // R,),
        in_specs=[
            pl.BlockSpec((R, D), lambda i: (i, 0)),
            pl.BlockSpec((R, D), lambda i: (i, 0)),
            pl.BlockSpec((D, D), lambda i: (0, 0)),
            pl.BlockSpec((D, D), lambda i: (0, 0)),
            pl.BlockSpec((D, D), lambda i: (0, 0)),
            pl.BlockSpec((1, D), lambda i: (0, 0)),
        ],
        out_specs=[
            pl.BlockSpec((R, D), lambda i: (i, 0)),
            pl.BlockSpec((R, D), lambda i: (i, 0)),
        ],
        out_shape=[
            jax.ShapeDtypeStruct((N_PAD, D), jnp.float32),
            jax.ShapeDtypeStruct((N_PAD, D), jnp.bfloat16),
        ],
    )(a, h, U1, U2, Wn, bn.reshape(1, D))


def _upd4_body(a_ref, h_ref, u1_ref, u2_ref, w4_ref, b4_ref, hn_ref, y4_ref):
    hn = jnp.maximum(
        lax.dot_general(a_ref[...].astype(jnp.float32), u1_ref[...], _DN,
                        preferred_element_type=jnp.float32)
        + lax.dot_general(h_ref[...], u2_ref[...], _DN,
                          preferred_element_type=jnp.float32),
        0.0,
    )
    hn_ref[...] = hn
    y4 = lax.dot_general(hn, w4_ref[...], _DN,
                         preferred_element_type=jnp.float32)
    y4_ref[...] = jnp.maximum(y4[:, 0] + b4_ref[0, 0], 0.0)


def _upd4(a, h, U1, U2, W4, b4):
    return pl.pallas_call(
        _upd4_body,
        grid=(N_PAD // R,),
        in_specs=[
            pl.BlockSpec((R, D), lambda i: (i, 0)),
            pl.BlockSpec((R, D), lambda i: (i, 0)),
            pl.BlockSpec((D, D), lambda i: (0, 0)),
            pl.BlockSpec((D, D), lambda i: (0, 0)),
            pl.BlockSpec((1, D), lambda i: (0, 0)),
            pl.BlockSpec((1, 1), lambda i: (0, 0)),
        ],
        out_specs=[
            pl.BlockSpec((R, D), lambda i: (i, 0)),
            pl.BlockSpec((R,), lambda i: (i,)),
        ],
        out_shape=[
            jax.ShapeDtypeStruct((N_PAD, D), jnp.float32),
            jax.ShapeDtypeStruct((N_PAD,), jnp.float32),
        ],
    )(a, h, U1, U2, W4, b4.reshape(1, 1))


def _final_body(a4_ref, h_ref, u4r_ref, u40_ref, o_ref):
    r = lax.dot_general(h_ref[...], u4r_ref[...], _DN,
                        preferred_element_type=jnp.float32)
    v = u40_ref[0, 0] * a4_ref[...] + r[:, 0]
    o_ref[...] = jax.nn.sigmoid(v)


def _final(a4, h, u4r, u40):
    return pl.pallas_call(
        _final_body,
        grid=(N_PAD // R,),
        in_specs=[
            pl.BlockSpec((R,), lambda i: (i,)),
            pl.BlockSpec((R, D), lambda i: (i, 0)),
            pl.BlockSpec((1, D), lambda i: (0, 0)),
            pl.BlockSpec((1, 1), lambda i: (0, 0)),
        ],
        out_specs=pl.BlockSpec((R,), lambda i: (i,)),
        out_shape=jax.ShapeDtypeStruct((N_PAD,), jnp.float32),
    )(a4, h, u4r, u40)


def _pack_rows(y_bf):
    return lax.bitcast_convert_type(
        y_bf.reshape(N_PAD // 2, D, 2), jnp.int32)


def _unpack_rows(a32):
    return lax.bitcast_convert_type(a32, jnp.bfloat16).reshape(N_PAD, D)


def kernel(x, edge_index, batch, lin_W1, lin_b1, upd_W1, lin_W2, lin_b2,
           upd_W2, lin_W3, lin_b3, upd_W3, lin_W4, lin_b4, upd_W4):
    src = edge_index[0]
    dst = edge_index[1]
    x_pad = jnp.zeros((N_PAD, D), jnp.float32).at[:N].set(x)

    buckets, counts = _bucket_kernel(src, dst)

    y1 = _pack_rows(_lin(x_pad, lin_W1, lin_b1))
    a1 = _unpack_rows(_segmax_kernel(y1, buckets, counts))
    h1, y2 = _upd(a1, x_pad, upd_W1[:, :D], upd_W1[:, D:], lin_W2, lin_b2)
    a2 = _unpack_rows(_segmax_kernel(_pack_rows(y2), buckets, counts))
    h2, y3 = _upd(a2, h1, upd_W2[:, :D], upd_W2[:, D:], lin_W3, lin_b3)
    a3 = _unpack_rows(_segmax_kernel(_pack_rows(y3), buckets, counts))
    h3, y4 = _upd4(a3, h2, upd_W3[:, :D], upd_W3[:, D:], lin_W4, lin_b4)
    a4 = _segmax1_kernel(y4, buckets, counts)
    out = _final(a4, h3, upd_W4[:, 1:], upd_W4[:, :1])
    return out[:N]


# speedup vs baseline: 2.6050x; 2.6050x over previous
import functools

import jax
import jax.numpy as jnp
from jax import lax
from jax.experimental import pallas as pl
from jax.experimental.pallas import tpu as pltpu
from jax.experimental.pallas import tpu_sc as plsc

N = 10000
E = 320000
D = 128

NW = 32
NPT = 320
N_PAD = NW * NPT

CH = 3200
FLUSH = 32768
BUF = FLUSH + CH + 16
K = 128
K4 = 256
NEG = -3.0e38
DW = D // 2
HN = N_PAD // 2
HT = NPT // 2

_mesh = plsc.VectorSubcoreMesh(
    core_axis_name="c", subcore_axis_name="s", num_cores=2, num_subcores=16
)
_SC_PARAMS = pltpu.CompilerParams(needs_layout_passes=False)


def _wid():
    return lax.axis_index("s") * 2 + lax.axis_index("c")


@functools.partial(
    pl.kernel,
    out_type=(
        jax.ShapeDtypeStruct((NW * E,), jnp.int32),
        jax.ShapeDtypeStruct((NW * 16,), jnp.int32),
    ),
    mesh=_mesh,
    compiler_params=_SC_PARAMS,
    scratch_types=[
        pltpu.VMEM((CH,), jnp.int32),
        pltpu.VMEM((CH,), jnp.int32),
        pltpu.VMEM((CH,), jnp.int32),
        pltpu.VMEM((CH,), jnp.int32),
        pltpu.VMEM((BUF,), jnp.int32),
        pltpu.VMEM((16,), jnp.int32),
        pltpu.SemaphoreType.DMA,
        pltpu.SemaphoreType.DMA,
    ],
)
def _bucket_kernel(src_hbm, dst_hbm, buckets_hbm, counts_hbm,
                   sbuf0, sbuf1, dbuf0, dbuf1, obuf, cntv, sem0, sem1):
    wid = _wid()
    lo = wid * NPT
    hi = lo + NPT
    one16 = jnp.full((16,), 1, jnp.int32)
    zero16 = jnp.full((16,), 0, jnp.int32)
    sems = (sem0, sem1)
    sbufs = (sbuf0, sbuf1)
    dbufs = (dbuf0, dbuf1)
    NCH = E // CH

    def start_load(ch, b):
        base = pl.multiple_of(ch * CH, 8)
        pltpu.async_copy(src_hbm.at[pl.ds(base, CH)], sbufs[b], sems[b])
        pltpu.async_copy(dst_hbm.at[pl.ds(base, CH)], dbufs[b], sems[b])

    def wait_load(ch, b):
        base = pl.multiple_of(ch * CH, 8)
        pltpu.make_async_copy(src_hbm.at[pl.ds(base, CH)], sbufs[b],
                              sems[b]).wait()
        pltpu.make_async_copy(dst_hbm.at[pl.ds(base, CH)], dbufs[b],
                              sems[b]).wait()

    start_load(0, 0)

    def do_chunk(ch, b, carry):
        @pl.when(ch + 1 < NCH)
        def _():
            start_load(ch + 1, 1 - b)

        wait_load(ch, b)

        def group_body(gg, carry):
            cnt, off = carry
            for u in range(4):
                g = gg * 4 + u
                d16 = dbufs[b][pl.ds(g * 16, 16)]
                s16 = sbufs[b][pl.ds(g * 16, 16)]
                m = (d16 >= lo) & (d16 < hi)
                mi = jnp.where(m, one16, zero16)
                ent = (s16 << 9) | (d16 - lo)
                cs = plsc.cumsum(mi)
                pos = cnt + cs - 1
                plsc.store_scatter(obuf, [pos], ent, mask=m)
                cnt = cnt + cs[15]
            return cnt, off

        cnt, off = lax.fori_loop(0, CH // 64, group_body, carry)

        do_flush = cnt >= FLUSH

        @pl.when(do_flush)
        def _():
            pltpu.sync_copy(
                obuf.at[pl.ds(0, FLUSH)],
                buckets_hbm.at[pl.ds(pl.multiple_of(wid * E + off, 8),
                                     FLUSH)],
            )
            nmv = (cnt - FLUSH + 15) >> 4

            def mv_body(i, _):
                obuf[pl.ds(i * 16, 16)] = obuf[pl.ds(FLUSH + i * 16, 16)]
                return 0

            lax.fori_loop(0, nmv, mv_body, 0)

        cnt = jnp.where(do_flush, cnt - FLUSH, cnt)
        off = jnp.where(do_flush, off + FLUSH, off)
        return cnt, off

    def pair_body(p, carry):
        for b in range(2):
            carry = do_chunk(p * 2 + b, b, carry)
        return carry

    cnt, off = lax.fori_loop(
        0, NCH // 2, pair_body, (jnp.int32(0), jnp.int32(0))
    )

    nblk = (cnt + 511) >> 9

    def blk_body(b, _):
        pltpu.sync_copy(
            obuf.at[pl.ds(pl.multiple_of(b * 512, 8), 512)],
            buckets_hbm.at[pl.ds(pl.multiple_of(wid * E + off + b * 512, 8),
                                 512)],
        )
        return 0

    lax.fori_loop(0, nblk, blk_body, 0)
    cntv[...] = jnp.full((16,), 0, jnp.int32) + (off + cnt)
    pltpu.sync_copy(cntv, counts_hbm.at[pl.ds(pl.multiple_of(wid * 16, 8),
                                              16)])


@functools.partial(
    pl.kernel,
    out_type=jax.ShapeDtypeStruct((N_PAD // 2, D), jnp.int32),
    mesh=_mesh,
    compiler_params=_SC_PARAMS,
    scratch_types=[
        pltpu.VMEM((K,), jnp.int32),
        pltpu.VMEM((K,), jnp.int32),
        pltpu.VMEM((K,), jnp.int32),
        pltpu.VMEM((K,), jnp.int32),
        pltpu.VMEM((K,), jnp.int32),
        pltpu.VMEM((K,), jnp.int32),
        pltpu.VMEM((K,), jnp.int32),
        pltpu.VMEM((K,), jnp.int32),
        pltpu.VMEM((K, D), jnp.int32),
        pltpu.VMEM((K, D), jnp.int32),
        pltpu.VMEM((NPT + 8, DW), jnp.int32),
        pltpu.VMEM((16,), jnp.int32),
        pltpu.SemaphoreType.DMA,
        pltpu.SemaphoreType.DMA,
        pltpu.SemaphoreType.DMA,
        pltpu.SemaphoreType.DMA,
    ],
)
def _segmax_kernel(y_hbm, buckets_hbm, counts_hbm, aggr_hbm,
                   pk0, pk1, sidx0, sidx1, dloc0, dloc1, soff0, soff1,
                   rows0, rows1, acc, cntv, gsem0, gsem1, psem0, psem1):
    wid = _wid()
    lo = wid * NPT
    iota = lax.iota(jnp.int32, 16)
    pks = (pk0, pk1)
    gsems = (gsem0, gsem1)
    psems = (psem0, psem1)
    sidxs = (sidx0, sidx1)
    dlocs = (dloc0, dloc1)
    soffs = (soff0, soff1)
    rowss = (rows0, rows1)
    pltpu.sync_copy(counts_hbm.at[pl.ds(pl.multiple_of(wid * 16, 8), 16)],
                    cntv)
    cnt = cntv[...][0]
    nchunks = (cnt + (K - 1)) // K

    def pk_slice(ch):
        return buckets_hbm.at[
            pl.ds(pl.multiple_of(wid * E + ch * K, 8), K)]

    def start_pk(ch, b):
        pltpu.async_copy(pk_slice(ch), pks[b], psems[b])

    def unpack_and_gather(ch, b):
        pltpu.make_async_copy(pk_slice(ch), pks[b], psems[b]).wait()
        base = ch * K
        for g in range(K // 16):
            e = pks[b][pl.ds(g * 16, 16)]
            valid = (base + g * 16 + iota) < cnt
            src = jnp.where(valid, e >> 9, jnp.full((16,), 0, jnp.int32))
            hi = src >= HN
            sp = jnp.where(hi, src - HN, src)
            so = jnp.where(hi, jnp.full((16,), DW, jnp.int32),
                           jnp.full((16,), 0, jnp.int32))
            dl = jnp.where(valid, e & 511, jnp.full((16,), NPT, jnp.int32))
            sl16 = pl.ds(g * 16, 16)
            sidxs[b][sl16] = sp
            soffs[b][sl16] = so
            dlocs[b][sl16] = dl
        pltpu.async_copy(y_hbm.at[sidxs[b]], rowss[b], gsems[b])

    @pl.when(nchunks > 0)
    def _():
        start_pk(0, 0)

    hs = DW * (wid // 16)
    lop = pl.multiple_of((wid % 16) * NPT, 8)
    for t in range(NPT // 64):
        pltpu.sync_copy(y_hbm.at[pl.ds(lop + t * 64, 64)],
                        rows0.at[pl.ds(0, 64)])

        def ini_body(r, _):
            for c in range(DW // 16):
                acc[t * 64 + r, pl.ds(c * 16, 16)] = (
                    rows0[r, pl.ds(hs + c * 16, 16)])
            return 0

        lax.fori_loop(0, 64, ini_body, 0)

    @pl.when(nchunks > 0)
    def _():
        unpack_and_gather(0, 0)

    @pl.when(nchunks > 1)
    def _():
        start_pk(1, 1)

    def do_chunk(ch, b):
        @pl.when(ch + 1 < nchunks)
        def _():
            unpack_and_gather(ch + 1, 1 - b)

        @pl.when(ch + 2 < nchunks)
        def _():
            start_pk(ch + 2, b)

        @pl.when(ch < nchunks)
        def _():
            pltpu.make_async_copy(y_hbm.at[sidxs[b]], rowss[b],
                                  gsems[b]).wait()
            for g in range(K // 16):
                sl16 = pl.ds(g * 16, 16)
                dl16 = dlocs[b][sl16]
                so16 = soffs[b][sl16]
                for lane in range(16):
                    d = dl16[lane]
                    so = so16[lane]
                    ei = g * 16 + lane
                    for c in range(DW // 16):
                        sla = pl.ds(c * 16, 16)
                        a = plsc.bitcast(acc[d, sla], jnp.bfloat16)
                        r = plsc.bitcast(
                            rowss[b][ei, pl.ds(so + c * 16, 16)],
                            jnp.bfloat16)
                        acc[d, sla] = plsc.bitcast(jnp.maximum(a, r),
                                                   jnp.int32)

    def pair_body(p, _):
        do_chunk(p * 2, 0)
        do_chunk(p * 2 + 1, 1)
        return 0

    lax.fori_loop(0, (nchunks + 1) // 2, pair_body, 0)

    for t in range(HT // 32):
        def wb_body(r, _):
            for c in range(DW // 16):
                rows0[r, pl.ds(c * 16, 16)] = (
                    acc[t * 32 + r, pl.ds(c * 16, 16)])
                rows0[r, pl.ds(DW + c * 16, 16)] = (
                    acc[HT + t * 32 + r, pl.ds(c * 16, 16)])
            return 0

        lax.fori_loop(0, 32, wb_body, 0)
        pltpu.sync_copy(
            rows0.at[pl.ds(0, 32)],
            aggr_hbm.at[pl.ds(pl.multiple_of(wid * HT + t * 32, 8), 32)])


@functools.partial(
    pl.kernel,
    out_type=jax.ShapeDtypeStruct((N_PAD,), jnp.float32),
    mesh=_mesh,
    compiler_params=_SC_PARAMS,
    scratch_types=[
        pltpu.VMEM((N_PAD,), jnp.float32),
        pltpu.VMEM((K4,), jnp.int32),
        pltpu.VMEM((16, 512), jnp.float32),
        pltpu.VMEM((NPT,), jnp.float32),
        pltpu.VMEM((16,), jnp.int32),
    ],
)
def _segmax1_kernel(y4_hbm, buckets_hbm, counts_hbm, aggr_hbm,
                    y4v, pk, acc2, outv, cntv):
    wid = _wid()
    lo = pl.multiple_of(wid * NPT, 8)
    iota = lax.iota(jnp.int32, 16)
    pltpu.sync_copy(y4_hbm, y4v)
    pltpu.sync_copy(counts_hbm.at[pl.ds(pl.multiple_of(wid * 16, 8), 16)],
                    cntv)
    cnt = cntv[...][0]
    neg = jnp.full((16,), NEG, jnp.float32)
    for r in range(16):
        for c in range(512 // 16):
            acc2[r, pl.ds(c * 16, 16)] = neg
    nchunks = (cnt + (K4 - 1)) // K4

    def chunk(ch, _):
        base = ch * K4
        pltpu.sync_copy(
            buckets_hbm.at[pl.ds(pl.multiple_of(wid * E + base, 8), K4)], pk)
        for g in range(K4 // 16):
            e = pk[pl.ds(g * 16, 16)]
            valid = (base + g * 16 + iota) < cnt
            src = jnp.where(valid, e >> 9, jnp.full((16,), 0, jnp.int32))
            dl = e & 511
            vals = plsc.load_gather(y4v, [src])
            cur = plsc.load_gather(acc2, [iota, dl])
            plsc.store_scatter(
                acc2, [iota, dl], jnp.maximum(vals, cur), mask=valid
            )
        return 0

    lax.fori_loop(0, nchunks, chunk, 0)
    for c in range(NPT // 16):
        v = y4v[pl.ds(lo + c * 16, 16)]
        for r in range(16):
            v = jnp.maximum(v, acc2[r, pl.ds(c * 16, 16)])
        outv[pl.ds(c * 16, 16)] = v
    pltpu.sync_copy(outv, aggr_hbm.at[pl.ds(lo, NPT)])


R = 1024
_DN = (((1,), (1,)), ((), ()))


def _lin_body(h_ref, w_ref, b_ref, y_ref):
    y_ref[...] = jnp.maximum(
        lax.dot_general(h_ref[...], w_ref[...], _DN,
                        preferred_element_type=jnp.float32) + b_ref[...],
        0.0,
    ).astype(jnp.bfloat16)


def _lin(h, W, b):
    return pl.pallas_call(
        _lin_body,
        grid=(N_PAD // R,),
        in_specs=[
            pl.BlockSpec((R, D), lambda i: (i, 0)),
            pl.BlockSpec((D, D), lambda i: (0, 0)),
            pl.BlockSpec((1, D), lambda i: (0, 0)),
        ],
        out_specs=pl.BlockSpec((R, D), lambda i: (i, 0)),
        out_shape=jax.ShapeDtypeStruct((N_PAD, D), jnp.bfloat16),
    )(h, W, b.reshape(1, D))


def _upd_body(a_ref, h_ref, u1_ref, u2_ref, w_ref, b_ref, hn_ref, yn_ref):
    hn = jnp.maximum(
        lax.dot_general(a_ref[...].astype(jnp.float32), u1_ref[...], _DN,
                        preferred_element_type=jnp.float32)
        + lax.dot_general(h_ref[...], u2_ref[...], _DN,
                          preferred_element_type=jnp.float32),
        0.0,
    )
    hn_ref[...] = hn
    yn_ref[...] = jnp.maximum(
        lax.dot_general(hn, w_ref[...], _DN,
                        preferred_element_type=jnp.float32) + b_ref[...],
        0.0,
    ).astype(jnp.bfloat16)


def _upd(a, h, U1, U2, Wn, bn):
    return pl.pallas_call(
        _upd_body,
        grid=(N_PAD // R,),
        in_specs=[
            pl.BlockSpec((R, D), lambda i: (i, 0)),
            pl.BlockSpec((R, D), lambda i: (i, 0)),
            pl.BlockSpec((D, D), lambda i: (0, 0)),
            pl.BlockSpec((D, D), lambda i: (0, 0)),
            pl.BlockSpec((D, D), lambda i: (0, 0)),
            pl.BlockSpec((1, D), lambda i: (0, 0)),
        ],
        out_specs=[
            pl.BlockSpec((R, D), lambda i: (i, 0)),
            pl.BlockSpec((R, D), lambda i: (i, 0)),
        ],
        out_shape=[
            jax.ShapeDtypeStruct((N_PAD, D), jnp.float32),
            jax.ShapeDtypeStruct((N_PAD, D), jnp.bfloat16),
        ],
    )(a, h, U1, U2, Wn, bn.reshape(1, D))


def _upd4_body(a_ref, h_ref, u1_ref, u2_ref, w4_ref, b4_ref, hn_ref, y4_ref):
    hn = jnp.maximum(
        lax.dot_general(a_ref[...].astype(jnp.float32), u1_ref[...], _DN,
                        preferred_element_type=jnp.float32)
        + lax.dot_general(h_ref[...], u2_ref[...], _DN,
                          preferred_element_type=jnp.float32),
        0.0,
    )
    hn_ref[...] = hn
    y4 = lax.dot_general(hn, w4_ref[...], _DN,
                         preferred_element_type=jnp.float32)
    y4_ref[...] = jnp.maximum(y4[:, 0] + b4_ref[0, 0], 0.0)


def _upd4(a, h, U1, U2, W4, b4):
    return pl.pallas_call(
        _upd4_body,
        grid=(N_PAD // R,),
        in_specs=[
            pl.BlockSpec((R, D), lambda i: (i, 0)),
            pl.BlockSpec((R, D), lambda i: (i, 0)),
            pl.BlockSpec((D, D), lambda i: (0, 0)),
            pl.BlockSpec((D, D), lambda i: (0, 0)),
            pl.BlockSpec((1, D), lambda i: (0, 0)),
            pl.BlockSpec((1, 1), lambda i: (0, 0)),
        ],
        out_specs=[
            pl.BlockSpec((R, D), lambda i: (i, 0)),
            pl.BlockSpec((R,), lambda i: (i,)),
        ],
        out_shape=[
            jax.ShapeDtypeStruct((N_PAD, D), jnp.float32),
            jax.ShapeDtypeStruct((N_PAD,), jnp.float32),
        ],
    )(a, h, U1, U2, W4, b4.reshape(1, 1))


def _final_body(a4_ref, h_ref, u4r_ref, u40_ref, o_ref):
    r = lax.dot_general(h_ref[...], u4r_ref[...], _DN,
                        preferred_element_type=jnp.float32)
    v = u40_ref[0, 0] * a4_ref[...] + r[:, 0]
    o_ref[...] = jax.nn.sigmoid(v)


def _final(a4, h, u4r, u40):
    return pl.pallas_call(
        _final_body,
        grid=(N_PAD // R,),
        in_specs=[
            pl.BlockSpec((R,), lambda i: (i,)),
            pl.BlockSpec((R, D), lambda i: (i, 0)),
            pl.BlockSpec((1, D), lambda i: (0, 0)),
            pl.BlockSpec((1, 1), lambda i: (0, 0)),
        ],
        out_specs=pl.BlockSpec((R,), lambda i: (i,)),
        out_shape=jax.ShapeDtypeStruct((N_PAD,), jnp.float32),
    )(a4, h, u4r, u40)


def _pack_rows(y_bf):
    u = lax.bitcast_convert_type(y_bf, jnp.uint16).astype(jnp.uint32)
    w = (u[:, DW:] << 16) | u[:, :DW]
    w = lax.bitcast_convert_type(w, jnp.int32)
    return jnp.concatenate([w[:HN], w[HN:]], axis=1)


def _unpack_rows(a32):
    u = lax.bitcast_convert_type(a32, jnp.uint32)

    def un(w):
        c_lo = lax.bitcast_convert_type(
            (w & jnp.uint32(0xFFFF)).astype(jnp.uint16), jnp.bfloat16)
        c_hi = lax.bitcast_convert_type(
            (w >> 16).astype(jnp.uint16), jnp.bfloat16)
        return jnp.concatenate([c_lo, c_hi], axis=1)

    a_rows = un(u[:, :DW]).reshape(NW, HT, D)
    b_rows = un(u[:, DW:]).reshape(NW, HT, D)
    return jnp.concatenate([a_rows, b_rows], axis=1).reshape(N_PAD, D)


def kernel(x, edge_index, batch, lin_W1, lin_b1, upd_W1, lin_W2, lin_b2,
           upd_W2, lin_W3, lin_b3, upd_W3, lin_W4, lin_b4, upd_W4):
    src = edge_index[0]
    dst = edge_index[1]
    x_pad = jnp.zeros((N_PAD, D), jnp.float32).at[:N].set(x)

    buckets, counts = _bucket_kernel(src, dst)

    y1 = _pack_rows(_lin(x_pad, lin_W1, lin_b1))
    a1 = _unpack_rows(_segmax_kernel(y1, buckets, counts))
    h1, y2 = _upd(a1, x_pad, upd_W1[:, :D], upd_W1[:, D:], lin_W2, lin_b2)
    a2 = _unpack_rows(_segmax_kernel(_pack_rows(y2), buckets, counts))
    h2, y3 = _upd(a2, h1, upd_W2[:, :D], upd_W2[:, D:], lin_W3, lin_b3)
    a3 = _unpack_rows(_segmax_kernel(_pack_rows(y3), buckets, counts))
    h3, y4 = _upd4(a3, h2, upd_W3[:, :D], upd_W3[:, D:], lin_W4, lin_b4)
    a4 = _segmax1_kernel(y4, buckets, counts)
    out = _final(a4, h3, upd_W4[:, 1:], upd_W4[:, :1])
    return out[:N]
